# baseline (speedup 1.0000x reference)
import functools

import jax
import jax.numpy as jnp
from jax.experimental import pallas as pl
from jax.experimental.pallas import tpu as pltpu

D = 128
ALPHA = 0.5


def _mm_bias_kernel(x_ref, w_ref, b_ref, o_ref):
    o_ref[...] = (
        jnp.dot(x_ref[...], w_ref[...], preferred_element_type=jnp.float32)
        + b_ref[...]
    )


def _mm_bias(x, w, b, tile):
    n = x.shape[0]
    return pl.pallas_call(
        _mm_bias_kernel,
        grid=(n // tile,),
        in_specs=[
            pl.BlockSpec((tile, D), lambda i: (i, 0)),
            pl.BlockSpec((D, D), lambda i: (0, 0)),
            pl.BlockSpec((1, D), lambda i: (0, 0)),
        ],
        out_specs=pl.BlockSpec((tile, D), lambda i: (i, 0)),
        out_shape=jax.ShapeDtypeStruct((n, D), jnp.float32),
        compiler_params=pltpu.CompilerParams(dimension_semantics=("parallel",)),
    )(x, w, b.reshape(1, D))


def _final_kernel(x_ref, x0_ref, m_ref, invc_ref, mask_ref, w2a_ref, w2b_ref,
                  b2_ref, ww_ref, wb_ref, o_ref):
    p = jnp.dot(x_ref[...], w2a_ref[...], preferred_element_type=jnp.float32)
    q = jnp.dot(m_ref[...] * invc_ref[...], w2b_ref[...],
                preferred_element_type=jnp.float32)
    xv = mask_ref[...] * (p + b2_ref[...]) + q
    pre = (1.0 - ALPHA) * xv + ALPHA * x0_ref[...]
    o_ref[...] = (
        jnp.dot(pre, ww_ref[...], preferred_element_type=jnp.float32)
        + wb_ref[...]
    )


def _final_stage(x, x0, msum, inv_cnt, mask, w2a, w2b, b2, ww, wb, tile=1000):
    n = x.shape[0]
    full = lambda i: (0, 0)
    row = lambda i: (i, 0)
    return pl.pallas_call(
        _final_kernel,
        grid=(n // tile,),
        in_specs=[
            pl.BlockSpec((tile, D), row),
            pl.BlockSpec((tile, D), row),
            pl.BlockSpec((tile, D), row),
            pl.BlockSpec((tile, 1), row),
            pl.BlockSpec((tile, 1), row),
            pl.BlockSpec((D, D), full),
            pl.BlockSpec((D, D), full),
            pl.BlockSpec((1, D), full),
            pl.BlockSpec((D, D), full),
            pl.BlockSpec((1, D), full),
        ],
        out_specs=pl.BlockSpec((tile, D), row),
        out_shape=jax.ShapeDtypeStruct((n, D), jnp.float32),
        compiler_params=pltpu.CompilerParams(dimension_semantics=("parallel",)),
    )(x, x0, msum, inv_cnt, mask, w2a, w2b, b2.reshape(1, D), ww, wb.reshape(1, D))


def kernel(X, edge_index, edge_attr, X0, W1_w, W1_b, W2_w, W2_b, W_w, W_b):
    N = X.shape[0]
    E = edge_attr.shape[0]
    H = N
    src = edge_index[0]
    dst = edge_index[1]

    W1a, W1b = W1_w[:D], W1_w[D:]
    W2a, W2b = W2_w[:D], W2_w[D:]

    XW1 = _mm_bias(X, W1a, jnp.zeros((D,), jnp.float32), tile=1000)
    EB = _mm_bias(edge_attr, W1b, W1_b, tile=2000)

    ean = EB + jnp.take(XW1, src, axis=0)

    ones = jnp.ones((E,), jnp.float32)
    cnt_dst = jax.ops.segment_sum(ones, dst, num_segments=H)
    cnt_src = jax.ops.segment_sum(ones, src, num_segments=N)

    xe_sum = jax.ops.segment_sum(ean, dst, num_segments=H)
    xe = xe_sum / jnp.maximum(cnt_dst, 1.0)[:, None]

    msum = jax.ops.segment_sum(jnp.take(xe, dst, axis=0), src, num_segments=N)
    inv_cnt = (1.0 / jnp.maximum(cnt_src, 1.0))[:, None]
    mask = (cnt_src > 0).astype(jnp.float32)[:, None]

    Xout = _final_stage(X, X0, msum, inv_cnt, mask, W2a, W2b, W2_b, W_w, W_b)
    return (Xout, ean, xe)


# speedup vs baseline: 4.0009x; 4.0009x over previous
import functools

import jax
import jax.numpy as jnp
from jax import lax
from jax.experimental import pallas as pl
from jax.experimental.pallas import tpu as pltpu
from jax.experimental.pallas import tpu_sc as plsc

D = 128
ALPHA = 0.5

NC = 2
NS = 16
L = 16
NW = NC * NS
BLK = 128


def _mm_bias_kernel(x_ref, w_ref, b_ref, o_ref):
    o_ref[...] = (
        jnp.dot(x_ref[...], w_ref[...], preferred_element_type=jnp.float32)
        + b_ref[...]
    )


def _mm_bias(x, w, b, tile):
    n = x.shape[0]
    return pl.pallas_call(
        _mm_bias_kernel,
        grid=(n // tile,),
        in_specs=[
            pl.BlockSpec((tile, D), lambda i: (i, 0)),
            pl.BlockSpec((D, D), lambda i: (0, 0)),
            pl.BlockSpec((1, D), lambda i: (0, 0)),
        ],
        out_specs=pl.BlockSpec((tile, D), lambda i: (i, 0)),
        out_shape=jax.ShapeDtypeStruct((n, D), jnp.float32),
        compiler_params=pltpu.CompilerParams(dimension_semantics=("parallel",)),
    )(x, w, b.reshape(1, D))


def _xe_norm_kernel(p0_ref, p1_ref, invd_ref, o_ref):
    o_ref[...] = (p0_ref[...] + p1_ref[...]) * invd_ref[...]


def _xe_norm(p0, p1, invd, tile):
    h = p0.shape[0]
    row = lambda i: (i, 0)
    return pl.pallas_call(
        _xe_norm_kernel,
        grid=(h // tile,),
        in_specs=[
            pl.BlockSpec((tile, D), row),
            pl.BlockSpec((tile, D), row),
            pl.BlockSpec((tile, 1), row),
        ],
        out_specs=pl.BlockSpec((tile, D), row),
        out_shape=jax.ShapeDtypeStruct((h, D), jnp.float32),
        compiler_params=pltpu.CompilerParams(dimension_semantics=("parallel",)),
    )(p0, p1, invd)


def _final_kernel(x_ref, x0_ref, m_ref, invc_ref, mask_ref, w2a_ref,
                  w2b_ref, b2_ref, ww_ref, wb_ref, o_ref):
    p = jnp.dot(x_ref[...], w2a_ref[...], preferred_element_type=jnp.float32)
    q = jnp.dot(m_ref[...] * invc_ref[...], w2b_ref[...],
                preferred_element_type=jnp.float32)
    xv = mask_ref[...] * (p + b2_ref[...]) + q
    pre = (1.0 - ALPHA) * xv + ALPHA * x0_ref[...]
    o_ref[...] = (
        jnp.dot(pre, ww_ref[...], preferred_element_type=jnp.float32)
        + wb_ref[...]
    )


def _final_stage(x, x0, m, inv_cnt, mask, w2a, w2b, b2, ww, wb, tile=1000):
    n = x.shape[0]
    full = lambda i: (0, 0)
    row = lambda i: (i, 0)
    return pl.pallas_call(
        _final_kernel,
        grid=(n // tile,),
        in_specs=[
            pl.BlockSpec((tile, D), row),
            pl.BlockSpec((tile, D), row),
            pl.BlockSpec((tile, D), row),
            pl.BlockSpec((tile, 1), row),
            pl.BlockSpec((tile, 1), row),
            pl.BlockSpec((D, D), full),
            pl.BlockSpec((D, D), full),
            pl.BlockSpec((1, D), full),
            pl.BlockSpec((D, D), full),
            pl.BlockSpec((1, D), full),
        ],
        out_specs=pl.BlockSpec((tile, D), row),
        out_shape=jax.ShapeDtypeStruct((n, D), jnp.float32),
        compiler_params=pltpu.CompilerParams(dimension_semantics=("parallel",)),
    )(x, x0, m, inv_cnt, mask, w2a, w2b, b2.reshape(1, D), ww, wb.reshape(1, D))


def _wid_blocks(wid, nblk):
    per = nblk // NW
    rem = nblk - per * NW
    base = wid * per + jnp.minimum(wid, rem)
    cnt = per + jnp.where(wid < rem, 1, 0)
    return base, cnt


def _pad16(n):
    per = -(-n // NS)
    return -(-per // 8) * 8


def _sc_edge_build(E, H):
    nblk = E // BLK
    rows_h = _pad16(H)
    hp = rows_h * NS
    mesh = plsc.VectorSubcoreMesh(core_axis_name="c", subcore_axis_name="s")

    @functools.partial(
        pl.kernel,
        out_type=[
            jax.ShapeDtypeStruct((E, D), jnp.float32),
            jax.ShapeDtypeStruct((NC, hp, D), jnp.float32),
        ],
        mesh=mesh,
        scratch_types=[
            pltpu.VMEM((BLK,), jnp.int32),
            pltpu.VMEM((1, BLK), jnp.int32),
            pltpu.VMEM((BLK, D), jnp.float32),
            pltpu.VMEM((BLK, D), jnp.float32),
            pltpu.VMEM_SHARED((hp, D), jnp.float32),
            pltpu.SemaphoreType.DMA,
            pltpu.SemaphoreType.DMA,
        ],
    )
    def sc_edge(eb_hbm, xw1_hbm, src_hbm, dst_hbm, zf_hbm,
                ean_hbm, xep_hbm,
                srcv, dstv, rowsE, rowsG, xe_acc, sem_e, sem_g):
        cid = lax.axis_index("c")
        sid = lax.axis_index("s")
        wid = sid * NC + cid

        rh0 = sid * rows_h
        pltpu.sync_copy(zf_hbm, xe_acc.at[pl.ds(rh0, rows_h)])
        plsc.subcore_barrier()

        base, cnt = _wid_blocks(wid, nblk)

        def do_block(j, c):
            b = base + j
            pltpu.sync_copy(src_hbm.at[b], srcv)
            pltpu.sync_copy(dst_hbm.at[b], dstv.at[0])
            ce = pltpu.async_copy(eb_hbm.at[pl.ds(b * BLK, BLK)], rowsE, sem_e)
            cg = pltpu.async_copy(xw1_hbm.at[srcv], rowsG, sem_g)
            ce.wait()
            cg.wait()

            def vadd(r, cc):
                for k in range(D // L):
                    s = pl.ds(k * L, L)
                    rowsE[r, s] = rowsE[r, s] + rowsG[r, s]
                return cc

            lax.fori_loop(0, BLK, vadd, 0)
            pltpu.sync_copy(rowsE, ean_hbm.at[pl.ds(b * BLK, BLK)])
            pltpu.sync_copy(rowsE, xe_acc.at[dstv.at[0]], add=True)
            return c

        lax.fori_loop(0, cnt, do_block, 0)

        plsc.subcore_barrier()
        pltpu.sync_copy(xe_acc.at[pl.ds(rh0, rows_h)],
                        xep_hbm.at[cid, pl.ds(rh0, rows_h)])

    return sc_edge


def _sc_e2v_build(E, H, N):
    nblk = E // BLK
    rows_n = _pad16(N)
    np_ = rows_n * NS
    mesh = plsc.VectorSubcoreMesh(core_axis_name="c", subcore_axis_name="s")

    @functools.partial(
        pl.kernel,
        out_type=jax.ShapeDtypeStruct((NC, np_, D), jnp.float32),
        mesh=mesh,
        scratch_types=[
            pltpu.VMEM((1, BLK), jnp.int32),
            pltpu.VMEM((BLK,), jnp.int32),
            pltpu.VMEM((BLK, D), jnp.float32),
            pltpu.VMEM_SHARED((np_, D), jnp.float32),
            pltpu.SemaphoreType.DMA,
        ],
    )
    def sc_e2v(xe_hbm, src_hbm, dst_hbm, zf_hbm,
               mp_hbm,
               srcv, dstv, rows, m_acc, sem_g):
        cid = lax.axis_index("c")
        sid = lax.axis_index("s")
        wid = sid * NC + cid

        rn0 = sid * rows_n
        pltpu.sync_copy(zf_hbm, m_acc.at[pl.ds(rn0, rows_n)])
        plsc.subcore_barrier()

        base, cnt = _wid_blocks(wid, nblk)

        def do_block(j, c):
            b = base + j
            pltpu.sync_copy(src_hbm.at[b], srcv.at[0])
            pltpu.sync_copy(dst_hbm.at[b], dstv)
            pltpu.async_copy(xe_hbm.at[dstv], rows, sem_g).wait()
            pltpu.sync_copy(rows, m_acc.at[srcv.at[0]], add=True)
            return c

        lax.fori_loop(0, cnt, do_block, 0)

        plsc.subcore_barrier()
        pltpu.sync_copy(m_acc.at[pl.ds(rn0, rows_n)],
                        mp_hbm.at[cid, pl.ds(rn0, rows_n)])

    return sc_e2v


def _sc_counts_build(E, H, N):
    nblk = E // BLK
    hp = _pad16(H) * NS
    np_ = _pad16(N) * NS
    mesh = plsc.VectorSubcoreMesh(core_axis_name="c", subcore_axis_name="s")

    @functools.partial(
        pl.kernel,
        out_type=[
            jax.ShapeDtypeStruct((NC, NS, hp), jnp.float32),
            jax.ShapeDtypeStruct((NC, NS, np_), jnp.float32),
        ],
        mesh=mesh,
        scratch_types=[
            pltpu.VMEM((BLK,), jnp.int32),
            pltpu.VMEM((BLK,), jnp.int32),
            pltpu.VMEM((hp,), jnp.float32),
            pltpu.VMEM((np_,), jnp.float32),
        ],
        compiler_params=pltpu.CompilerParams(needs_layout_passes=False),
    )
    def sc_counts(src_hbm, dst_hbm,
                  cdp_hbm, csp_hbm,
                  srcv, dstv, cd_hist, cs_hist):
        cid = lax.axis_index("c")
        sid = lax.axis_index("s")
        wid = sid * NC + cid

        zvec = jnp.zeros((L,), jnp.float32)

        def zero_h(i, c):
            cd_hist[pl.ds(i * L, L)] = zvec
            return c

        def zero_n(i, c):
            cs_hist[pl.ds(i * L, L)] = zvec
            return c

        lax.fori_loop(0, hp // L, zero_h, 0)
        lax.fori_loop(0, np_ // L, zero_n, 0)

        base, cnt = _wid_blocks(wid, nblk)
        onev = jnp.ones((L,), jnp.float32)

        def do_block(j, c):
            b = base + j
            pltpu.sync_copy(src_hbm.at[b], srcv)
            pltpu.sync_copy(dst_hbm.at[b], dstv)
            for k in range(BLK // L):
                s = pl.ds(k * L, L)
                plsc.addupdate_scatter(cd_hist, [dstv[s]], onev)
                plsc.addupdate_scatter(cs_hist, [srcv[s]], onev)
            return c

        lax.fori_loop(0, cnt, do_block, 0)

        pltpu.sync_copy(cd_hist, cdp_hbm.at[cid, sid])
        pltpu.sync_copy(cs_hist, csp_hbm.at[cid, sid])

    return sc_counts


def kernel(X, edge_index, edge_attr, X0, W1_w, W1_b, W2_w, W2_b, W_w, W_b):
    N = X.shape[0]
    E = edge_attr.shape[0]
    H = N
    src = edge_index[0]
    dst = edge_index[1]
    src2d = src.reshape(E // BLK, BLK)
    dst2d = dst.reshape(E // BLK, BLK)

    W1a, W1b = W1_w[:D], W1_w[D:]
    W2a, W2b = W2_w[:D], W2_w[D:]

    XW1 = _mm_bias(X, W1a, jnp.zeros((D,), jnp.float32), tile=1000)
    EB = _mm_bias(edge_attr, W1b, W1_b, tile=2000)

    cdp, csp = _sc_counts_build(E, H, N)(src2d, dst2d)
    cnt_dst = cdp.sum((0, 1))
    cnt_src = csp.sum((0, 1))[:N]

    zf = jnp.zeros((_pad16(H), D), jnp.float32)
    EB2 = lax.optimization_barrier((EB, cdp))[0]
    ean, xep = _sc_edge_build(E, H)(EB2, XW1, src2d, dst2d, zf)

    inv_d = (1.0 / jnp.maximum(cnt_dst, 1.0))[:, None]
    xe_pad = _xe_norm(xep[0], xep[1], inv_d, tile=_pad16(H))
    xe = xe_pad[:H]

    mp = _sc_e2v_build(E, H, N)(xe_pad, src2d, dst2d, zf)
    msum = mp[0, :N] + mp[1, :N]

    inv_c = (1.0 / jnp.maximum(cnt_src, 1.0))[:, None]
    mask = (cnt_src > 0).astype(jnp.float32)[:, None]

    Xout = _final_stage(X, X0, msum, inv_c, mask, W2a, W2b, W2_b, W_w, W_b)
    return (Xout, ean, xe)


# speedup vs baseline: 4.6652x; 1.1661x over previous
import functools

import jax
import jax.numpy as jnp
from jax import lax
from jax.experimental import pallas as pl
from jax.experimental.pallas import tpu as pltpu
from jax.experimental.pallas import tpu_sc as plsc

D = 128
ALPHA = 0.5

NC = 2
NS = 16
L = 16
NW = NC * NS
BLK = 128


def _mm_bias_kernel(x_ref, w_ref, b_ref, o_ref):
    o_ref[...] = (
        jnp.dot(x_ref[...], w_ref[...], preferred_element_type=jnp.float32)
        + b_ref[...]
    )


def _mm_bias(x, w, b, tile):
    n = x.shape[0]
    return pl.pallas_call(
        _mm_bias_kernel,
        grid=(n // tile,),
        in_specs=[
            pl.BlockSpec((tile, D), lambda i: (i, 0)),
            pl.BlockSpec((D, D), lambda i: (0, 0)),
            pl.BlockSpec((1, D), lambda i: (0, 0)),
        ],
        out_specs=pl.BlockSpec((tile, D), lambda i: (i, 0)),
        out_shape=jax.ShapeDtypeStruct((n, D), jnp.float32),
        compiler_params=pltpu.CompilerParams(dimension_semantics=("parallel",)),
    )(x, w, b.reshape(1, D))


def _xe_norm_kernel(p0_ref, p1_ref, invd_ref, o_ref):
    o_ref[...] = (p0_ref[...] + p1_ref[...]) * invd_ref[...]


def _xe_norm(p0, p1, invd, tile):
    h = p0.shape[0]
    row = lambda i: (i, 0)
    return pl.pallas_call(
        _xe_norm_kernel,
        grid=(h // tile,),
        in_specs=[
            pl.BlockSpec((tile, D), row),
            pl.BlockSpec((tile, D), row),
            pl.BlockSpec((tile, 1), row),
        ],
        out_specs=pl.BlockSpec((tile, D), row),
        out_shape=jax.ShapeDtypeStruct((h, D), jnp.float32),
        compiler_params=pltpu.CompilerParams(dimension_semantics=("parallel",)),
    )(p0, p1, invd)


def _final_kernel(x_ref, x0_ref, m_ref, invc_ref, mask_ref, w2a_ref,
                  w2b_ref, b2_ref, ww_ref, wb_ref, o_ref):
    p = jnp.dot(x_ref[...], w2a_ref[...], preferred_element_type=jnp.float32)
    q = jnp.dot(m_ref[...] * invc_ref[...], w2b_ref[...],
                preferred_element_type=jnp.float32)
    xv = mask_ref[...] * (p + b2_ref[...]) + q
    pre = (1.0 - ALPHA) * xv + ALPHA * x0_ref[...]
    o_ref[...] = (
        jnp.dot(pre, ww_ref[...], preferred_element_type=jnp.float32)
        + wb_ref[...]
    )


def _final_stage(x, x0, m, inv_cnt, mask, w2a, w2b, b2, ww, wb, tile=1000):
    n = x.shape[0]
    full = lambda i: (0, 0)
    row = lambda i: (i, 0)
    return pl.pallas_call(
        _final_kernel,
        grid=(n // tile,),
        in_specs=[
            pl.BlockSpec((tile, D), row),
            pl.BlockSpec((tile, D), row),
            pl.BlockSpec((tile, D), row),
            pl.BlockSpec((tile, 1), row),
            pl.BlockSpec((tile, 1), row),
            pl.BlockSpec((D, D), full),
            pl.BlockSpec((D, D), full),
            pl.BlockSpec((1, D), full),
            pl.BlockSpec((D, D), full),
            pl.BlockSpec((1, D), full),
        ],
        out_specs=pl.BlockSpec((tile, D), row),
        out_shape=jax.ShapeDtypeStruct((n, D), jnp.float32),
        compiler_params=pltpu.CompilerParams(dimension_semantics=("parallel",)),
    )(x, x0, m, inv_cnt, mask, w2a, w2b, b2.reshape(1, D), ww, wb.reshape(1, D))


def _wid_blocks(wid, nblk):
    per = nblk // NW
    rem = nblk - per * NW
    base = wid * per + jnp.minimum(wid, rem)
    cnt = per + jnp.where(wid < rem, 1, 0)
    return base, cnt


def _pad16(n):
    per = -(-n // NS)
    return -(-per // 8) * 8


def _sc_edge_build(E, H):
    nblk = E // BLK
    rows_h = _pad16(H)
    hp = rows_h * NS
    mesh = plsc.VectorSubcoreMesh(core_axis_name="c", subcore_axis_name="s")

    @functools.partial(
        pl.kernel,
        out_type=[
            jax.ShapeDtypeStruct((E, D), jnp.float32),
            jax.ShapeDtypeStruct((NC, hp, D), jnp.float32),
        ],
        mesh=mesh,
        scratch_types=[
            pltpu.VMEM((BLK,), jnp.int32),
            pltpu.VMEM((1, BLK), jnp.int32),
            pltpu.VMEM((BLK, D), jnp.float32),
            pltpu.VMEM((BLK, D), jnp.float32),
            pltpu.VMEM_SHARED((hp, D), jnp.float32),
            pltpu.SemaphoreType.DMA,
            pltpu.SemaphoreType.DMA,
            pltpu.SemaphoreType.DMA,
            pltpu.SemaphoreType.DMA,
        ],
    )
    def sc_edge(eb_hbm, xw1_hbm, src_hbm, dst_hbm, zf_hbm,
                ean_hbm, xep_hbm,
                srcv, dstv, rowsE, rowsG, xe_acc,
                sem_e, sem_g, sem_w1, sem_w2):
        cid = lax.axis_index("c")
        sid = lax.axis_index("s")
        wid = sid * NC + cid

        rh0 = sid * rows_h
        pltpu.sync_copy(zf_hbm, xe_acc.at[pl.ds(rh0, rows_h)])
        plsc.subcore_barrier()

        base, cnt = _wid_blocks(wid, nblk)

        def do_block(j, c):
            b = base + j
            pltpu.sync_copy(src_hbm.at[b], srcv)
            ce = pltpu.async_copy(eb_hbm.at[pl.ds(b * BLK, BLK)], rowsE, sem_e)
            cg = pltpu.async_copy(xw1_hbm.at[srcv], rowsG, sem_g)
            pltpu.sync_copy(dst_hbm.at[b], dstv.at[0])
            ce.wait()
            cg.wait()

            def vadd(r, cc):
                for k in range(D // L):
                    s = pl.ds(k * L, L)
                    rowsE[r, s] = rowsE[r, s] + rowsG[r, s]
                return cc

            lax.fori_loop(0, BLK, vadd, 0)
            w1 = pltpu.async_copy(rowsE, ean_hbm.at[pl.ds(b * BLK, BLK)],
                                  sem_w1)
            w2 = pltpu.async_copy(rowsE, xe_acc.at[dstv.at[0]], sem_w2,
                                  add=True)
            w1.wait()
            w2.wait()
            return c

        lax.fori_loop(0, cnt, do_block, 0)

        plsc.subcore_barrier()
        pltpu.sync_copy(xe_acc.at[pl.ds(rh0, rows_h)],
                        xep_hbm.at[cid, pl.ds(rh0, rows_h)])

    return sc_edge


def _sc_e2v_build(E, H, N):
    nblk = E // BLK
    rows_n = _pad16(N)
    np_ = rows_n * NS
    mesh = plsc.VectorSubcoreMesh(core_axis_name="c", subcore_axis_name="s")

    @functools.partial(
        pl.kernel,
        out_type=jax.ShapeDtypeStruct((NC, np_, D), jnp.float32),
        mesh=mesh,
        scratch_types=[
            pltpu.VMEM((2, BLK), jnp.int32),
            pltpu.VMEM((BLK,), jnp.int32),
            pltpu.VMEM((BLK,), jnp.int32),
            pltpu.VMEM((BLK, D), jnp.float32),
            pltpu.VMEM((BLK, D), jnp.float32),
            pltpu.VMEM_SHARED((np_, D), jnp.float32),
            pltpu.SemaphoreType.DMA,
            pltpu.SemaphoreType.DMA,
        ],
    )
    def sc_e2v(xe_hbm, src_hbm, dst_hbm, zf_hbm,
               mp_hbm,
               srcv, dstv0, dstv1, rows0, rows1, m_acc, sem_g0, sem_g1):
        cid = lax.axis_index("c")
        sid = lax.axis_index("s")
        wid = sid * NC + cid

        rn0 = sid * rows_n
        pltpu.sync_copy(zf_hbm, m_acc.at[pl.ds(rn0, rows_n)])
        plsc.subcore_barrier()

        base, cnt = _wid_blocks(wid, nblk)
        npairs = cnt // 2

        def do_pair(j, c):
            b0 = base + 2 * j
            b1 = b0 + 1
            pltpu.sync_copy(dst_hbm.at[b0], dstv0)
            pltpu.sync_copy(dst_hbm.at[b1], dstv1)
            g0 = pltpu.async_copy(xe_hbm.at[dstv0], rows0, sem_g0)
            g1 = pltpu.async_copy(xe_hbm.at[dstv1], rows1, sem_g1)
            pltpu.sync_copy(src_hbm.at[b0], srcv.at[0])
            pltpu.sync_copy(src_hbm.at[b1], srcv.at[1])
            g0.wait()
            pltpu.sync_copy(rows0, m_acc.at[srcv.at[0]], add=True)
            g1.wait()
            pltpu.sync_copy(rows1, m_acc.at[srcv.at[1]], add=True)
            return c

        lax.fori_loop(0, npairs, do_pair, 0)

        def do_single(j, c):
            b = base + 2 * npairs + j
            pltpu.sync_copy(dst_hbm.at[b], dstv0)
            g0 = pltpu.async_copy(xe_hbm.at[dstv0], rows0, sem_g0)
            pltpu.sync_copy(src_hbm.at[b], srcv.at[0])
            g0.wait()
            pltpu.sync_copy(rows0, m_acc.at[srcv.at[0]], add=True)
            return c

        lax.fori_loop(0, cnt - 2 * npairs, do_single, 0)

        plsc.subcore_barrier()
        pltpu.sync_copy(m_acc.at[pl.ds(rn0, rows_n)],
                        mp_hbm.at[cid, pl.ds(rn0, rows_n)])

    return sc_e2v


def _sc_counts_build(E, H, N):
    nblk = E // BLK
    hp = _pad16(H) * NS
    np_ = _pad16(N) * NS
    mesh = plsc.VectorSubcoreMesh(core_axis_name="c", subcore_axis_name="s")

    @functools.partial(
        pl.kernel,
        out_type=[
            jax.ShapeDtypeStruct((NC, NS, hp), jnp.float32),
            jax.ShapeDtypeStruct((NC, NS, np_), jnp.float32),
        ],
        mesh=mesh,
        scratch_types=[
            pltpu.VMEM((BLK,), jnp.int32),
            pltpu.VMEM((BLK,), jnp.int32),
            pltpu.VMEM((hp,), jnp.float32),
            pltpu.VMEM((np_,), jnp.float32),
        ],
        compiler_params=pltpu.CompilerParams(needs_layout_passes=False),
    )
    def sc_counts(src_hbm, dst_hbm,
                  cdp_hbm, csp_hbm,
                  srcv, dstv, cd_hist, cs_hist):
        cid = lax.axis_index("c")
        sid = lax.axis_index("s")
        wid = sid * NC + cid

        zvec = jnp.zeros((L,), jnp.float32)

        def zero_h(i, c):
            cd_hist[pl.ds(i * L, L)] = zvec
            return c

        def zero_n(i, c):
            cs_hist[pl.ds(i * L, L)] = zvec
            return c

        lax.fori_loop(0, hp // L, zero_h, 0)
        lax.fori_loop(0, np_ // L, zero_n, 0)

        base, cnt = _wid_blocks(wid, nblk)
        onev = jnp.ones((L,), jnp.float32)

        def do_block(j, c):
            b = base + j
            pltpu.sync_copy(src_hbm.at[b], srcv)
            pltpu.sync_copy(dst_hbm.at[b], dstv)
            for k in range(BLK // L):
                s = pl.ds(k * L, L)
                plsc.addupdate_scatter(cd_hist, [dstv[s]], onev)
                plsc.addupdate_scatter(cs_hist, [srcv[s]], onev)
            return c

        lax.fori_loop(0, cnt, do_block, 0)

        pltpu.sync_copy(cd_hist, cdp_hbm.at[cid, sid])
        pltpu.sync_copy(cs_hist, csp_hbm.at[cid, sid])

    return sc_counts


def kernel(X, edge_index, edge_attr, X0, W1_w, W1_b, W2_w, W2_b, W_w, W_b):
    N = X.shape[0]
    E = edge_attr.shape[0]
    H = N
    src = edge_index[0]
    dst = edge_index[1]
    src2d = src.reshape(E // BLK, BLK)
    dst2d = dst.reshape(E // BLK, BLK)

    W1a, W1b = W1_w[:D], W1_w[D:]
    W2a, W2b = W2_w[:D], W2_w[D:]

    XW1 = _mm_bias(X, W1a, jnp.zeros((D,), jnp.float32), tile=1000)
    EB = _mm_bias(edge_attr, W1b, W1_b, tile=2000)

    cdp, csp = _sc_counts_build(E, H, N)(src2d, dst2d)
    cnt_dst = cdp.sum((0, 1))
    cnt_src = csp.sum((0, 1))[:N]

    zf = jnp.zeros((_pad16(H), D), jnp.float32)
    EB2 = lax.optimization_barrier((EB, cdp))[0]
    ean, xep = _sc_edge_build(E, H)(EB2, XW1, src2d, dst2d, zf)

    inv_d = (1.0 / jnp.maximum(cnt_dst, 1.0))[:, None]
    xe_pad = _xe_norm(xep[0], xep[1], inv_d, tile=_pad16(H))
    xe = xe_pad[:H]

    mp = _sc_e2v_build(E, H, N)(xe_pad, src2d, dst2d, zf)
    msum = mp[0, :N] + mp[1, :N]

    inv_c = (1.0 / jnp.maximum(cnt_src, 1.0))[:, None]
    mask = (cnt_src > 0).astype(jnp.float32)[:, None]

    Xout = _final_stage(X, X0, msum, inv_c, mask, W2a, W2b, W2_b, W_w, W_b)
    return (Xout, ean, xe)
